# baseline (speedup 1.0000x reference)
import functools

import jax
import jax.numpy as jnp
from jax.experimental import pallas as pl
from jax.experimental.pallas import tpu as pltpu

NHEADS = 8


def _prep_scan_body(x_ref, g_ref, b_ref, wi_ref, bi_ref, wdt_ref, bdt_ref,
                    wb_ref, bb_ref, wc_ref, bc_ref, alogT_ref, dskip_ref,
                    y_ref, h_ref, da_ref, db_ref, hs_ref, *, chunk):
    t = pl.program_id(1)

    @pl.when(t == 0)
    def _():
        h_ref[...] = jnp.zeros_like(h_ref)

    x = x_ref[0]
    m = jnp.mean(x, axis=-1, keepdims=True)
    xc = x - m
    v = jnp.mean(xc * xc, axis=-1, keepdims=True)
    xn = xc * jax.lax.rsqrt(v + 1e-5) * g_ref[...] + b_ref[...]
    u = jnp.dot(xn, wi_ref[...], preferred_element_type=jnp.float32) + bi_ref[...]
    dt = jax.nn.softplus(
        jnp.dot(u, wdt_ref[...], preferred_element_type=jnp.float32) + bdt_ref[...])
    bm = jnp.dot(u, wb_ref[...], preferred_element_type=jnp.float32) + bb_ref[...]
    cm = jnp.dot(u, wc_ref[...], preferred_element_type=jnp.float32) + bc_ref[...]

    a_neg = -jnp.exp(alogT_ref[...])
    da_ref[...] = jnp.exp(dt[:, None, :] * a_neg[None, :, :])
    db_ref[...] = (dt * u)[:, None, :] * bm[:, :, None]

    def step(i, h):
        h = da_ref[i] * h + db_ref[i]
        hs_ref[i] = h
        return h

    h = jax.lax.fori_loop(0, chunk, step, h_ref[...])
    h_ref[...] = h
    y = jnp.sum(hs_ref[...] * cm[:, :, None], axis=1)
    y_ref[0] = y + u * dskip_ref[...]


def _qkv_body(y_ref, w_ref, b_ref, o_ref, *, hd):
    res = jnp.dot(y_ref[...], w_ref[...],
                  preferred_element_type=jnp.float32) + b_ref[...]
    for j in range(res.shape[-1] // hd):
        o_ref[0, j] = res[:, j * hd:(j + 1) * hd]


def _attn_epi_body(q_ref, k_ref, v_ref, y_ref, wo_ref, bo_ref, ang_ref,
                   anb_ref, wout_ref, bout_ref, o_ref, ps_ref, acc_ref, *,
                   scale, hd):
    i = pl.program_id(1)
    acc_ref[...] = jnp.zeros_like(acc_ref)

    def head(h, _):
        q = q_ref[0, h]
        k = k_ref[0, h]
        vv = v_ref[0, h]
        s = jax.lax.dot_general(q, k, (((1,), (1,)), ((), ())),
                                preferred_element_type=jnp.float32) * scale
        mx = jnp.max(s, axis=-1, keepdims=True)
        p = jnp.exp(s - mx)
        l = jnp.sum(p, axis=-1, keepdims=True)
        o = jnp.dot(p, vv, preferred_element_type=jnp.float32) / l
        wo_h = wo_ref[pl.ds(pl.multiple_of(h * hd, hd), hd), :]
        acc_ref[...] += jnp.dot(o, wo_h, preferred_element_type=jnp.float32)
        return 0

    jax.lax.fori_loop(0, NHEADS, head, 0)
    a = acc_ref[...] + bo_ref[...]
    m = jnp.mean(a, axis=-1, keepdims=True)
    ac = a - m
    v = jnp.mean(ac * ac, axis=-1, keepdims=True)
    ln = ac * jax.lax.rsqrt(v + 1e-5) * ang_ref[...] + anb_ref[...]
    z = y_ref[0] + ln
    out = jnp.dot(z, wout_ref[...], preferred_element_type=jnp.float32) + bout_ref[...]
    o_ref[0] = out
    ps = jnp.sum(out, axis=0, keepdims=True)

    @pl.when(i == 0)
    def _():
        ps_ref[0] = ps

    @pl.when(i != 0)
    def _():
        ps_ref[0] = ps_ref[0] + ps


def _cls_body(pa_ref, pv_ref, g_ref, b_ref, o_ref, *, inv_count):
    s = (pa_ref[0] + pv_ref[0]) * inv_count
    m = jnp.mean(s, axis=-1, keepdims=True)
    sc = s - m
    v = jnp.mean(sc * sc, axis=-1, keepdims=True)
    o_ref[0] = sc * jax.lax.rsqrt(v + 1e-5) * g_ref[...] + b_ref[...]


def kernel(text, audio, video, in_norm_g, in_norm_b, Wi, bi, Wdt, bdt, WB, bB,
           WC, bC, A_log, Dskip, Wqkv, bqkv, Wo, bo, an_g, an_b, Wout, bout,
           on_g, on_b):
    f32 = jnp.float32
    nb, lc, d = text.shape
    n = A_log.shape[1]
    hd = d // NHEADS
    g = 2 * nb
    l = 2 * lc
    rows = g * l

    def fit(feat):
        lm = feat.shape[1]
        if lm < lc:
            feat = jnp.pad(feat, ((0, 0), (0, lc - lm), (0, 0)))
        elif lm > lc:
            feat = feat[:, :lc]
        return feat

    xa = jnp.concatenate([fit(audio), text], axis=1)
    xv = jnp.concatenate([fit(video), text], axis=1)
    x4 = jnp.concatenate([xa, xv], axis=0)

    row = lambda a: a.reshape(1, -1)
    full = lambda shape: pl.BlockSpec(shape, lambda *a: (0,) * len(shape))

    q = 128
    nt = l // q
    y4 = pl.pallas_call(
        functools.partial(_prep_scan_body, chunk=q),
        grid=(g, nt),
        in_specs=[
            pl.BlockSpec((1, q, d), lambda b, t: (b, t, 0)),
            full((1, d)), full((1, d)),
            full((d, d)), full((1, d)),
            full((d, d)), full((1, d)),
            full((d, n)), full((1, n)),
            full((d, n)), full((1, n)),
            full((n, d)), full((1, d)),
        ],
        out_specs=pl.BlockSpec((1, q, d), lambda b, t: (b, t, 0)),
        out_shape=jax.ShapeDtypeStruct((g, l, d), f32),
        scratch_shapes=[
            pltpu.VMEM((n, d), f32),
            pltpu.VMEM((q, n, d), f32),
            pltpu.VMEM((q, n, d), f32),
            pltpu.VMEM((q, n, d), f32),
        ],
        compiler_params=pltpu.CompilerParams(
            dimension_semantics=(pltpu.PARALLEL, pltpu.ARBITRARY),
            vmem_limit_bytes=100 * 1024 * 1024),
    )(x4, row(in_norm_g), row(in_norm_b), Wi.T, row(bi), Wdt.T, row(bdt),
      WB.T, row(bB), WC.T, row(bC), A_log.T, row(Dskip))

    rb = 1024
    nrb = l // rb
    y2 = y4.reshape(rows, d)
    qkvh = pl.pallas_call(
        functools.partial(_qkv_body, hd=hd),
        grid=(rows // rb,),
        in_specs=[
            pl.BlockSpec((rb, d), lambda r: (r, 0)),
            full((d, 3 * d)), full((1, 3 * d)),
        ],
        out_specs=pl.BlockSpec((1, 3 * NHEADS, rb, hd),
                               lambda r: (r // nrb, 0, r % nrb, 0)),
        out_shape=jax.ShapeDtypeStruct((g, 3 * NHEADS, l, hd), f32),
        compiler_params=pltpu.CompilerParams(
            dimension_semantics=(pltpu.PARALLEL,)),
    )(y2, Wqkv.T, row(bqkv))

    qb = 1024
    nqb = l // qb
    out4, psum = pl.pallas_call(
        functools.partial(_attn_epi_body, scale=1.0 / float(hd) ** 0.5, hd=hd),
        grid=(g, nqb),
        in_specs=[
            pl.BlockSpec((1, NHEADS, qb, hd), lambda b, i: (b, 0, i, 0)),
            pl.BlockSpec((1, NHEADS, l, hd), lambda b, i: (b, 1, 0, 0)),
            pl.BlockSpec((1, NHEADS, l, hd), lambda b, i: (b, 2, 0, 0)),
            pl.BlockSpec((1, qb, d), lambda b, i: (b, i, 0)),
            full((d, d)), full((1, d)), full((1, d)), full((1, d)),
            full((d, d)), full((1, d)),
        ],
        out_specs=[
            pl.BlockSpec((1, qb, d), lambda b, i: (b, i, 0)),
            pl.BlockSpec((1, 1, d), lambda b, i: (b, 0, 0)),
        ],
        out_shape=[
            jax.ShapeDtypeStruct((g, l, d), f32),
            jax.ShapeDtypeStruct((g, 1, d), f32),
        ],
        scratch_shapes=[pltpu.VMEM((qb, d), f32)],
        compiler_params=pltpu.CompilerParams(
            dimension_semantics=(pltpu.PARALLEL, pltpu.ARBITRARY),
            vmem_limit_bytes=100 * 1024 * 1024),
    )(qkvh, qkvh, qkvh, y4, Wo.T, row(bo), row(an_g), row(an_b),
      Wout.T, row(bout))

    cls = pl.pallas_call(
        functools.partial(_cls_body, inv_count=0.5 / float(l)),
        grid=(nb,),
        in_specs=[
            pl.BlockSpec((1, 1, d), lambda b: (b, 0, 0)),
            pl.BlockSpec((1, 1, d), lambda b: (b + nb, 0, 0)),
            full((1, d)), full((1, d)),
        ],
        out_specs=pl.BlockSpec((1, 1, d), lambda b: (b, 0, 0)),
        out_shape=jax.ShapeDtypeStruct((nb, 1, d), f32),
        compiler_params=pltpu.CompilerParams(
            dimension_semantics=(pltpu.ARBITRARY,)),
    )(psum, psum, row(on_g), row(on_b))

    return cls.reshape(nb, d), out4[:nb], out4[nb:]


# speedup vs baseline: 1.0092x; 1.0092x over previous
import functools

import jax
import jax.numpy as jnp
from jax.experimental import pallas as pl
from jax.experimental.pallas import tpu as pltpu

NHEADS = 8


def _prep_scan_body(x_ref, g_ref, b_ref, wi_ref, bi_ref, wdt_ref, bdt_ref,
                    wb_ref, bb_ref, wc_ref, bc_ref, alogT_ref, dskip_ref,
                    y_ref, h_ref, da_ref, db_ref, hs_ref, *, chunk):
    t = pl.program_id(1)

    @pl.when(t == 0)
    def _():
        h_ref[...] = jnp.zeros_like(h_ref)

    x = x_ref[0]
    m = jnp.mean(x, axis=-1, keepdims=True)
    xc = x - m
    v = jnp.mean(xc * xc, axis=-1, keepdims=True)
    xn = xc * jax.lax.rsqrt(v + 1e-5) * g_ref[...] + b_ref[...]
    u = jnp.dot(xn, wi_ref[...], preferred_element_type=jnp.float32) + bi_ref[...]
    dt = jax.nn.softplus(
        jnp.dot(u, wdt_ref[...], preferred_element_type=jnp.float32) + bdt_ref[...])
    bm = jnp.dot(u, wb_ref[...], preferred_element_type=jnp.float32) + bb_ref[...]
    cm = jnp.dot(u, wc_ref[...], preferred_element_type=jnp.float32) + bc_ref[...]

    a_neg = -jnp.exp(alogT_ref[...])
    da_ref[...] = jnp.exp(dt[:, None, :] * a_neg[None, :, :])
    db_ref[...] = (dt * u)[:, None, :] * bm[:, :, None]

    def step(i, h):
        h = da_ref[i] * h + db_ref[i]
        hs_ref[i] = h
        return h

    h = jax.lax.fori_loop(0, chunk, step, h_ref[...])
    h_ref[...] = h
    y = jnp.sum(hs_ref[...] * cm[:, :, None], axis=1)
    y_ref[0] = y + u * dskip_ref[...]


def _qkv_body(y_ref, w_ref, b_ref, o_ref, *, hd):
    res = jnp.dot(y_ref[...], w_ref[...],
                  preferred_element_type=jnp.float32) + b_ref[...]
    for j in range(res.shape[-1] // hd):
        o_ref[0, j] = res[:, j * hd:(j + 1) * hd]


def _attn_epi_body(q_ref, k_ref, v_ref, y_ref, wo_ref, bo_ref, ang_ref,
                   anb_ref, wout_ref, bout_ref, o_ref, ps_ref, acc_ref, *,
                   scale, hd):
    i = pl.program_id(1)
    acc_ref[...] = jnp.zeros_like(acc_ref)

    def head_pair(hp, _):
        for dh in range(2):
            h = hp * 2 + dh
            q = q_ref[0, h]
            k = k_ref[0, h]
            vv = v_ref[0, h]
            s = jax.lax.dot_general(q, k, (((1,), (1,)), ((), ())),
                                    preferred_element_type=jnp.float32) * scale
            mx = jnp.max(s, axis=-1, keepdims=True)
            p = jnp.exp(s - mx)
            l = jnp.sum(p, axis=-1, keepdims=True)
            o = jnp.dot(p, vv, preferred_element_type=jnp.float32) / l
            wo_h = wo_ref[pl.ds(pl.multiple_of(h * hd, hd), hd), :]
            acc_ref[...] += jnp.dot(o, wo_h, preferred_element_type=jnp.float32)
        return 0

    jax.lax.fori_loop(0, NHEADS // 2, head_pair, 0)
    a = acc_ref[...] + bo_ref[...]
    m = jnp.mean(a, axis=-1, keepdims=True)
    ac = a - m
    v = jnp.mean(ac * ac, axis=-1, keepdims=True)
    ln = ac * jax.lax.rsqrt(v + 1e-5) * ang_ref[...] + anb_ref[...]
    z = y_ref[0] + ln
    out = jnp.dot(z, wout_ref[...], preferred_element_type=jnp.float32) + bout_ref[...]
    o_ref[0] = out
    ps = jnp.sum(out, axis=0, keepdims=True)

    @pl.when(i == 0)
    def _():
        ps_ref[0] = ps

    @pl.when(i != 0)
    def _():
        ps_ref[0] = ps_ref[0] + ps


def _cls_body(pa_ref, pv_ref, g_ref, b_ref, o_ref, *, inv_count):
    s = (pa_ref[0] + pv_ref[0]) * inv_count
    m = jnp.mean(s, axis=-1, keepdims=True)
    sc = s - m
    v = jnp.mean(sc * sc, axis=-1, keepdims=True)
    o_ref[0] = sc * jax.lax.rsqrt(v + 1e-5) * g_ref[...] + b_ref[...]


def kernel(text, audio, video, in_norm_g, in_norm_b, Wi, bi, Wdt, bdt, WB, bB,
           WC, bC, A_log, Dskip, Wqkv, bqkv, Wo, bo, an_g, an_b, Wout, bout,
           on_g, on_b):
    f32 = jnp.float32
    nb, lc, d = text.shape
    n = A_log.shape[1]
    hd = d // NHEADS
    g = 2 * nb
    l = 2 * lc
    rows = g * l

    def fit(feat):
        lm = feat.shape[1]
        if lm < lc:
            feat = jnp.pad(feat, ((0, 0), (0, lc - lm), (0, 0)))
        elif lm > lc:
            feat = feat[:, :lc]
        return feat

    xa = jnp.concatenate([fit(audio), text], axis=1)
    xv = jnp.concatenate([fit(video), text], axis=1)
    x4 = jnp.concatenate([xa, xv], axis=0)

    row = lambda a: a.reshape(1, -1)
    full = lambda shape: pl.BlockSpec(shape, lambda *a: (0,) * len(shape))

    q = 128
    nt = l // q
    y4 = pl.pallas_call(
        functools.partial(_prep_scan_body, chunk=q),
        grid=(g, nt),
        in_specs=[
            pl.BlockSpec((1, q, d), lambda b, t: (b, t, 0)),
            full((1, d)), full((1, d)),
            full((d, d)), full((1, d)),
            full((d, d)), full((1, d)),
            full((d, n)), full((1, n)),
            full((d, n)), full((1, n)),
            full((n, d)), full((1, d)),
        ],
        out_specs=pl.BlockSpec((1, q, d), lambda b, t: (b, t, 0)),
        out_shape=jax.ShapeDtypeStruct((g, l, d), f32),
        scratch_shapes=[
            pltpu.VMEM((n, d), f32),
            pltpu.VMEM((q, n, d), f32),
            pltpu.VMEM((q, n, d), f32),
            pltpu.VMEM((q, n, d), f32),
        ],
        compiler_params=pltpu.CompilerParams(
            dimension_semantics=(pltpu.PARALLEL, pltpu.ARBITRARY),
            vmem_limit_bytes=100 * 1024 * 1024),
    )(x4, row(in_norm_g), row(in_norm_b), Wi.T, row(bi), Wdt.T, row(bdt),
      WB.T, row(bB), WC.T, row(bC), A_log.T, row(Dskip))

    rb = 1024
    nrb = l // rb
    y2 = y4.reshape(rows, d)
    qkvh = pl.pallas_call(
        functools.partial(_qkv_body, hd=hd),
        grid=(rows // rb,),
        in_specs=[
            pl.BlockSpec((rb, d), lambda r: (r, 0)),
            full((d, 3 * d)), full((1, 3 * d)),
        ],
        out_specs=pl.BlockSpec((1, 3 * NHEADS, rb, hd),
                               lambda r: (r // nrb, 0, r % nrb, 0)),
        out_shape=jax.ShapeDtypeStruct((g, 3 * NHEADS, l, hd), f32),
        compiler_params=pltpu.CompilerParams(
            dimension_semantics=(pltpu.PARALLEL,)),
    )(y2, Wqkv.T, row(bqkv))

    qb = 1024
    nqb = l // qb
    out4, psum = pl.pallas_call(
        functools.partial(_attn_epi_body, scale=1.0 / float(hd) ** 0.5, hd=hd),
        grid=(g, nqb),
        in_specs=[
            pl.BlockSpec((1, NHEADS, qb, hd), lambda b, i: (b, 0, i, 0)),
            pl.BlockSpec((1, NHEADS, l, hd), lambda b, i: (b, 1, 0, 0)),
            pl.BlockSpec((1, NHEADS, l, hd), lambda b, i: (b, 2, 0, 0)),
            pl.BlockSpec((1, qb, d), lambda b, i: (b, i, 0)),
            full((d, d)), full((1, d)), full((1, d)), full((1, d)),
            full((d, d)), full((1, d)),
        ],
        out_specs=[
            pl.BlockSpec((1, qb, d), lambda b, i: (b, i, 0)),
            pl.BlockSpec((1, 1, d), lambda b, i: (b, 0, 0)),
        ],
        out_shape=[
            jax.ShapeDtypeStruct((g, l, d), f32),
            jax.ShapeDtypeStruct((g, 1, d), f32),
        ],
        scratch_shapes=[pltpu.VMEM((qb, d), f32)],
        compiler_params=pltpu.CompilerParams(
            dimension_semantics=(pltpu.PARALLEL, pltpu.ARBITRARY),
            vmem_limit_bytes=100 * 1024 * 1024),
    )(qkvh, qkvh, qkvh, y4, Wo.T, row(bo), row(an_g), row(an_b),
      Wout.T, row(bout))

    cls = pl.pallas_call(
        functools.partial(_cls_body, inv_count=0.5 / float(l)),
        grid=(nb,),
        in_specs=[
            pl.BlockSpec((1, 1, d), lambda b: (b, 0, 0)),
            pl.BlockSpec((1, 1, d), lambda b: (b + nb, 0, 0)),
            full((1, d)), full((1, d)),
        ],
        out_specs=pl.BlockSpec((1, 1, d), lambda b: (b, 0, 0)),
        out_shape=jax.ShapeDtypeStruct((nb, 1, d), f32),
        compiler_params=pltpu.CompilerParams(
            dimension_semantics=(pltpu.ARBITRARY,)),
    )(psum, psum, row(on_g), row(on_b))

    return cls.reshape(nb, d), out4[:nb], out4[nb:]


# speedup vs baseline: 1.0329x; 1.0234x over previous
import functools

import jax
import jax.numpy as jnp
from jax.experimental import pallas as pl
from jax.experimental.pallas import tpu as pltpu

NHEADS = 8


def _prep_scan_body(x_ref, g_ref, b_ref, wi_ref, bi_ref, wdt_ref, bdt_ref,
                    wb_ref, bb_ref, wc_ref, bc_ref, alogT_ref, dskip_ref,
                    y_ref, h_ref, da_ref, db_ref, hs_ref, *, chunk):
    t = pl.program_id(1)

    @pl.when(t == 0)
    def _():
        h_ref[...] = jnp.zeros_like(h_ref)

    x = x_ref[0]
    m = jnp.mean(x, axis=-1, keepdims=True)
    xc = x - m
    v = jnp.mean(xc * xc, axis=-1, keepdims=True)
    xn = xc * jax.lax.rsqrt(v + 1e-5) * g_ref[...] + b_ref[...]
    u = jnp.dot(xn, wi_ref[...], preferred_element_type=jnp.float32) + bi_ref[...]
    dt = jax.nn.softplus(
        jnp.dot(u, wdt_ref[...], preferred_element_type=jnp.float32) + bdt_ref[...])
    bm = jnp.dot(u, wb_ref[...], preferred_element_type=jnp.float32) + bb_ref[...]
    cm = jnp.dot(u, wc_ref[...], preferred_element_type=jnp.float32) + bc_ref[...]

    a_neg = -jnp.exp(alogT_ref[...])
    da_ref[...] = jnp.exp(dt[:, None, :] * a_neg[None, :, :])
    db_ref[...] = (dt * u)[:, None, :] * bm[:, :, None]

    def step(i, h):
        h = da_ref[i] * h + db_ref[i]
        hs_ref[i] = h
        return h

    h = jax.lax.fori_loop(0, chunk, step, h_ref[...])
    h_ref[...] = h
    y = jnp.sum(hs_ref[...] * cm[:, :, None], axis=1)
    y_ref[0] = y + u * dskip_ref[...]


def _qkv_body(y_ref, w_ref, b_ref, o_ref, *, hd):
    res = jnp.dot(y_ref[...], w_ref[...],
                  preferred_element_type=jnp.float32) + b_ref[...]
    for j in range(res.shape[-1] // hd):
        o_ref[0, j] = res[:, j * hd:(j + 1) * hd]


def _attn_epi_body(q_ref, k_ref, v_ref, y_ref, wo_ref, bo_ref, ang_ref,
                   anb_ref, wout_ref, bout_ref, o_ref, ps_ref, acc_ref, *,
                   scale, hd):
    i = pl.program_id(1)
    acc_ref[...] = jnp.zeros_like(acc_ref)

    def head_pair(hp, _):
        for dh in range(2):
            h = hp * 2 + dh
            q = q_ref[0, h]
            k = k_ref[0, h]
            vv = v_ref[0, h]
            s = jax.lax.dot_general(q, k, (((1,), (1,)), ((), ())),
                                    preferred_element_type=jnp.float32) * scale
            mx = jnp.max(s, axis=-1, keepdims=True)
            p = jnp.exp(s - mx)
            l = jnp.sum(p, axis=-1, keepdims=True)
            o = jnp.dot(p, vv, preferred_element_type=jnp.float32) / l
            wo_h = wo_ref[pl.ds(pl.multiple_of(h * hd, hd), hd), :]
            acc_ref[...] += jnp.dot(o, wo_h, preferred_element_type=jnp.float32)
        return 0

    jax.lax.fori_loop(0, NHEADS // 2, head_pair, 0)
    a = acc_ref[...] + bo_ref[...]
    m = jnp.mean(a, axis=-1, keepdims=True)
    ac = a - m
    v = jnp.mean(ac * ac, axis=-1, keepdims=True)
    ln = ac * jax.lax.rsqrt(v + 1e-5) * ang_ref[...] + anb_ref[...]
    z = y_ref[0] + ln
    out = jnp.dot(z, wout_ref[...], preferred_element_type=jnp.float32) + bout_ref[...]
    o_ref[0] = out
    ps = jnp.sum(out, axis=0, keepdims=True)

    @pl.when(i == 0)
    def _():
        ps_ref[0] = ps

    @pl.when(i != 0)
    def _():
        ps_ref[0] = ps_ref[0] + ps


def _cls_body(pa_ref, pv_ref, g_ref, b_ref, o_ref, *, inv_count):
    s = (pa_ref[0] + pv_ref[0]) * inv_count
    m = jnp.mean(s, axis=-1, keepdims=True)
    sc = s - m
    v = jnp.mean(sc * sc, axis=-1, keepdims=True)
    o_ref[0] = sc * jax.lax.rsqrt(v + 1e-5) * g_ref[...] + b_ref[...]


def kernel(text, audio, video, in_norm_g, in_norm_b, Wi, bi, Wdt, bdt, WB, bB,
           WC, bC, A_log, Dskip, Wqkv, bqkv, Wo, bo, an_g, an_b, Wout, bout,
           on_g, on_b):
    f32 = jnp.float32
    nb, lc, d = text.shape
    n = A_log.shape[1]
    hd = d // NHEADS
    g = 2 * nb
    l = 2 * lc
    rows = g * l

    def fit(feat):
        lm = feat.shape[1]
        if lm < lc:
            feat = jnp.pad(feat, ((0, 0), (0, lc - lm), (0, 0)))
        elif lm > lc:
            feat = feat[:, :lc]
        return feat

    xa = jnp.concatenate([fit(audio), text], axis=1)
    xv = jnp.concatenate([fit(video), text], axis=1)
    x4 = jnp.concatenate([xa, xv], axis=0)

    row = lambda a: a.reshape(1, -1)
    full = lambda shape: pl.BlockSpec(shape, lambda *a: (0,) * len(shape))

    q = 256
    nt = l // q
    y4 = pl.pallas_call(
        functools.partial(_prep_scan_body, chunk=q),
        grid=(g, nt),
        in_specs=[
            pl.BlockSpec((1, q, d), lambda b, t: (b, t, 0)),
            full((1, d)), full((1, d)),
            full((d, d)), full((1, d)),
            full((d, d)), full((1, d)),
            full((d, n)), full((1, n)),
            full((d, n)), full((1, n)),
            full((n, d)), full((1, d)),
        ],
        out_specs=pl.BlockSpec((1, q, d), lambda b, t: (b, t, 0)),
        out_shape=jax.ShapeDtypeStruct((g, l, d), f32),
        scratch_shapes=[
            pltpu.VMEM((n, d), f32),
            pltpu.VMEM((q, n, d), f32),
            pltpu.VMEM((q, n, d), f32),
            pltpu.VMEM((q, n, d), f32),
        ],
        compiler_params=pltpu.CompilerParams(
            dimension_semantics=(pltpu.PARALLEL, pltpu.ARBITRARY),
            vmem_limit_bytes=100 * 1024 * 1024),
    )(x4, row(in_norm_g), row(in_norm_b), Wi.T, row(bi), Wdt.T, row(bdt),
      WB.T, row(bB), WC.T, row(bC), A_log.T, row(Dskip))

    rb = 1024
    nrb = l // rb
    y2 = y4.reshape(rows, d)
    qkvh = pl.pallas_call(
        functools.partial(_qkv_body, hd=hd),
        grid=(rows // rb,),
        in_specs=[
            pl.BlockSpec((rb, d), lambda r: (r, 0)),
            full((d, 3 * d)), full((1, 3 * d)),
        ],
        out_specs=pl.BlockSpec((1, 3 * NHEADS, rb, hd),
                               lambda r: (r // nrb, 0, r % nrb, 0)),
        out_shape=jax.ShapeDtypeStruct((g, 3 * NHEADS, l, hd), f32),
        compiler_params=pltpu.CompilerParams(
            dimension_semantics=(pltpu.PARALLEL,)),
    )(y2, Wqkv.T, row(bqkv))

    qb = 1024
    nqb = l // qb
    out4, psum = pl.pallas_call(
        functools.partial(_attn_epi_body, scale=1.0 / float(hd) ** 0.5, hd=hd),
        grid=(g, nqb),
        in_specs=[
            pl.BlockSpec((1, NHEADS, qb, hd), lambda b, i: (b, 0, i, 0)),
            pl.BlockSpec((1, NHEADS, l, hd), lambda b, i: (b, 1, 0, 0)),
            pl.BlockSpec((1, NHEADS, l, hd), lambda b, i: (b, 2, 0, 0)),
            pl.BlockSpec((1, qb, d), lambda b, i: (b, i, 0)),
            full((d, d)), full((1, d)), full((1, d)), full((1, d)),
            full((d, d)), full((1, d)),
        ],
        out_specs=[
            pl.BlockSpec((1, qb, d), lambda b, i: (b, i, 0)),
            pl.BlockSpec((1, 1, d), lambda b, i: (b, 0, 0)),
        ],
        out_shape=[
            jax.ShapeDtypeStruct((g, l, d), f32),
            jax.ShapeDtypeStruct((g, 1, d), f32),
        ],
        scratch_shapes=[pltpu.VMEM((qb, d), f32)],
        compiler_params=pltpu.CompilerParams(
            dimension_semantics=(pltpu.PARALLEL, pltpu.ARBITRARY),
            vmem_limit_bytes=100 * 1024 * 1024),
    )(qkvh, qkvh, qkvh, y4, Wo.T, row(bo), row(an_g), row(an_b),
      Wout.T, row(bout))

    cls = pl.pallas_call(
        functools.partial(_cls_body, inv_count=0.5 / float(l)),
        grid=(nb,),
        in_specs=[
            pl.BlockSpec((1, 1, d), lambda b: (b, 0, 0)),
            pl.BlockSpec((1, 1, d), lambda b: (b + nb, 0, 0)),
            full((1, d)), full((1, d)),
        ],
        out_specs=pl.BlockSpec((1, 1, d), lambda b: (b, 0, 0)),
        out_shape=jax.ShapeDtypeStruct((nb, 1, d), f32),
        compiler_params=pltpu.CompilerParams(
            dimension_semantics=(pltpu.ARBITRARY,)),
    )(psum, psum, row(on_g), row(on_b))

    return cls.reshape(nb, d), out4[:nb], out4[nb:]


# speedup vs baseline: 1.0797x; 1.0454x over previous
import functools

import jax
import jax.numpy as jnp
from jax.experimental import pallas as pl
from jax.experimental.pallas import tpu as pltpu

NHEADS = 8


def _prep_scan_body(x_ref, g_ref, b_ref, wi_ref, bi_ref, wdt_ref, bdt_ref,
                    wb_ref, bb_ref, wc_ref, bc_ref, alogT_ref, dskip_ref,
                    y_ref, h_ref, da_ref, db_ref, hs_ref, *, chunk):
    t = pl.program_id(1)

    @pl.when(t == 0)
    def _():
        h_ref[...] = jnp.zeros_like(h_ref)

    x = x_ref[0]
    m = jnp.mean(x, axis=-1, keepdims=True)
    xc = x - m
    v = jnp.mean(xc * xc, axis=-1, keepdims=True)
    xn = xc * jax.lax.rsqrt(v + 1e-5) * g_ref[...] + b_ref[...]
    u = jnp.dot(xn, wi_ref[...], preferred_element_type=jnp.float32) + bi_ref[...]
    dt = jax.nn.softplus(
        jnp.dot(u, wdt_ref[...], preferred_element_type=jnp.float32) + bdt_ref[...])
    bm = jnp.dot(u, wb_ref[...], preferred_element_type=jnp.float32) + bb_ref[...]
    cm = jnp.dot(u, wc_ref[...], preferred_element_type=jnp.float32) + bc_ref[...]

    a_neg = -jnp.exp(alogT_ref[...])
    da_ref[...] = jnp.exp(dt[:, None, :] * a_neg[None, :, :])
    db_ref[...] = (dt * u)[:, None, :] * bm[:, :, None]

    def step(i, h):
        h = da_ref[i] * h + db_ref[i]
        hs_ref[i] = h
        return h

    h = jax.lax.fori_loop(0, chunk, step, h_ref[...], unroll=8)
    h_ref[...] = h
    y = jnp.sum(hs_ref[...] * cm[:, :, None], axis=1)
    y_ref[0] = y + u * dskip_ref[...]


def _qkv_body(y_ref, w_ref, b_ref, o_ref, *, hd):
    res = jnp.dot(y_ref[...], w_ref[...],
                  preferred_element_type=jnp.float32) + b_ref[...]
    for j in range(res.shape[-1] // hd):
        o_ref[0, j] = res[:, j * hd:(j + 1) * hd]


def _attn_epi_body(q_ref, k_ref, v_ref, y_ref, wo_ref, bo_ref, ang_ref,
                   anb_ref, wout_ref, bout_ref, o_ref, ps_ref, acc_ref, *,
                   scale, hd):
    i = pl.program_id(1)
    acc_ref[...] = jnp.zeros_like(acc_ref)

    def head_pair(hp, _):
        for dh in range(2):
            h = hp * 2 + dh
            q = q_ref[0, h]
            k = k_ref[0, h]
            vv = v_ref[0, h]
            s = jax.lax.dot_general(q, k, (((1,), (1,)), ((), ())),
                                    preferred_element_type=jnp.float32) * scale
            mx = jnp.max(s, axis=-1, keepdims=True)
            p = jnp.exp(s - mx)
            l = jnp.sum(p, axis=-1, keepdims=True)
            o = jnp.dot(p, vv, preferred_element_type=jnp.float32) / l
            wo_h = wo_ref[pl.ds(pl.multiple_of(h * hd, hd), hd), :]
            acc_ref[...] += jnp.dot(o, wo_h, preferred_element_type=jnp.float32)
        return 0

    jax.lax.fori_loop(0, NHEADS // 2, head_pair, 0)
    a = acc_ref[...] + bo_ref[...]
    m = jnp.mean(a, axis=-1, keepdims=True)
    ac = a - m
    v = jnp.mean(ac * ac, axis=-1, keepdims=True)
    ln = ac * jax.lax.rsqrt(v + 1e-5) * ang_ref[...] + anb_ref[...]
    z = y_ref[0] + ln
    out = jnp.dot(z, wout_ref[...], preferred_element_type=jnp.float32) + bout_ref[...]
    o_ref[0] = out
    ps = jnp.sum(out, axis=0, keepdims=True)

    @pl.when(i == 0)
    def _():
        ps_ref[0] = ps

    @pl.when(i != 0)
    def _():
        ps_ref[0] = ps_ref[0] + ps


def _cls_body(pa_ref, pv_ref, g_ref, b_ref, o_ref, *, inv_count):
    s = (pa_ref[0] + pv_ref[0]) * inv_count
    m = jnp.mean(s, axis=-1, keepdims=True)
    sc = s - m
    v = jnp.mean(sc * sc, axis=-1, keepdims=True)
    o_ref[0] = sc * jax.lax.rsqrt(v + 1e-5) * g_ref[...] + b_ref[...]


def kernel(text, audio, video, in_norm_g, in_norm_b, Wi, bi, Wdt, bdt, WB, bB,
           WC, bC, A_log, Dskip, Wqkv, bqkv, Wo, bo, an_g, an_b, Wout, bout,
           on_g, on_b):
    f32 = jnp.float32
    nb, lc, d = text.shape
    n = A_log.shape[1]
    hd = d // NHEADS
    g = 2 * nb
    l = 2 * lc
    rows = g * l

    def fit(feat):
        lm = feat.shape[1]
        if lm < lc:
            feat = jnp.pad(feat, ((0, 0), (0, lc - lm), (0, 0)))
        elif lm > lc:
            feat = feat[:, :lc]
        return feat

    xa = jnp.concatenate([fit(audio), text], axis=1)
    xv = jnp.concatenate([fit(video), text], axis=1)
    x4 = jnp.concatenate([xa, xv], axis=0)

    row = lambda a: a.reshape(1, -1)
    full = lambda shape: pl.BlockSpec(shape, lambda *a: (0,) * len(shape))

    q = 256
    nt = l // q
    y4 = pl.pallas_call(
        functools.partial(_prep_scan_body, chunk=q),
        grid=(g, nt),
        in_specs=[
            pl.BlockSpec((1, q, d), lambda b, t: (b, t, 0)),
            full((1, d)), full((1, d)),
            full((d, d)), full((1, d)),
            full((d, d)), full((1, d)),
            full((d, n)), full((1, n)),
            full((d, n)), full((1, n)),
            full((n, d)), full((1, d)),
        ],
        out_specs=pl.BlockSpec((1, q, d), lambda b, t: (b, t, 0)),
        out_shape=jax.ShapeDtypeStruct((g, l, d), f32),
        scratch_shapes=[
            pltpu.VMEM((n, d), f32),
            pltpu.VMEM((q, n, d), f32),
            pltpu.VMEM((q, n, d), f32),
            pltpu.VMEM((q, n, d), f32),
        ],
        compiler_params=pltpu.CompilerParams(
            dimension_semantics=(pltpu.PARALLEL, pltpu.ARBITRARY),
            vmem_limit_bytes=100 * 1024 * 1024),
    )(x4, row(in_norm_g), row(in_norm_b), Wi.T, row(bi), Wdt.T, row(bdt),
      WB.T, row(bB), WC.T, row(bC), A_log.T, row(Dskip))

    rb = 1024
    nrb = l // rb
    y2 = y4.reshape(rows, d)
    qkvh = pl.pallas_call(
        functools.partial(_qkv_body, hd=hd),
        grid=(rows // rb,),
        in_specs=[
            pl.BlockSpec((rb, d), lambda r: (r, 0)),
            full((d, 3 * d)), full((1, 3 * d)),
        ],
        out_specs=pl.BlockSpec((1, 3 * NHEADS, rb, hd),
                               lambda r: (r // nrb, 0, r % nrb, 0)),
        out_shape=jax.ShapeDtypeStruct((g, 3 * NHEADS, l, hd), f32),
        compiler_params=pltpu.CompilerParams(
            dimension_semantics=(pltpu.PARALLEL,)),
    )(y2, Wqkv.T, row(bqkv))

    qb = 1024
    nqb = l // qb
    out4, psum = pl.pallas_call(
        functools.partial(_attn_epi_body, scale=1.0 / float(hd) ** 0.5, hd=hd),
        grid=(g, nqb),
        in_specs=[
            pl.BlockSpec((1, NHEADS, qb, hd), lambda b, i: (b, 0, i, 0)),
            pl.BlockSpec((1, NHEADS, l, hd), lambda b, i: (b, 1, 0, 0)),
            pl.BlockSpec((1, NHEADS, l, hd), lambda b, i: (b, 2, 0, 0)),
            pl.BlockSpec((1, qb, d), lambda b, i: (b, i, 0)),
            full((d, d)), full((1, d)), full((1, d)), full((1, d)),
            full((d, d)), full((1, d)),
        ],
        out_specs=[
            pl.BlockSpec((1, qb, d), lambda b, i: (b, i, 0)),
            pl.BlockSpec((1, 1, d), lambda b, i: (b, 0, 0)),
        ],
        out_shape=[
            jax.ShapeDtypeStruct((g, l, d), f32),
            jax.ShapeDtypeStruct((g, 1, d), f32),
        ],
        scratch_shapes=[pltpu.VMEM((qb, d), f32)],
        compiler_params=pltpu.CompilerParams(
            dimension_semantics=(pltpu.PARALLEL, pltpu.ARBITRARY),
            vmem_limit_bytes=100 * 1024 * 1024),
    )(qkvh, qkvh, qkvh, y4, Wo.T, row(bo), row(an_g), row(an_b),
      Wout.T, row(bout))

    cls = pl.pallas_call(
        functools.partial(_cls_body, inv_count=0.5 / float(l)),
        grid=(nb,),
        in_specs=[
            pl.BlockSpec((1, 1, d), lambda b: (b, 0, 0)),
            pl.BlockSpec((1, 1, d), lambda b: (b + nb, 0, 0)),
            full((1, d)), full((1, d)),
        ],
        out_specs=pl.BlockSpec((1, 1, d), lambda b: (b, 0, 0)),
        out_shape=jax.ShapeDtypeStruct((nb, 1, d), f32),
        compiler_params=pltpu.CompilerParams(
            dimension_semantics=(pltpu.ARBITRARY,)),
    )(psum, psum, row(on_g), row(on_b))

    return cls.reshape(nb, d), out4[:nb], out4[nb:]


# speedup vs baseline: 1.0847x; 1.0046x over previous
import functools

import jax
import jax.numpy as jnp
from jax.experimental import pallas as pl
from jax.experimental.pallas import tpu as pltpu

NHEADS = 8


def _prep_scan_body(x_ref, g_ref, b_ref, wi_ref, bi_ref, wdt_ref, bdt_ref,
                    wb_ref, bb_ref, wc_ref, bc_ref, alogT_ref, dskip_ref,
                    y_ref, h_ref, da_ref, db_ref, hs_ref, *, chunk):
    t = pl.program_id(1)

    @pl.when(t == 0)
    def _():
        h_ref[...] = jnp.zeros_like(h_ref)

    x = x_ref[0]
    m = jnp.mean(x, axis=-1, keepdims=True)
    xc = x - m
    v = jnp.mean(xc * xc, axis=-1, keepdims=True)
    xn = xc * jax.lax.rsqrt(v + 1e-5) * g_ref[...] + b_ref[...]
    u = jnp.dot(xn, wi_ref[...], preferred_element_type=jnp.float32) + bi_ref[...]
    dt = jax.nn.softplus(
        jnp.dot(u, wdt_ref[...], preferred_element_type=jnp.float32) + bdt_ref[...])
    bm = jnp.dot(u, wb_ref[...], preferred_element_type=jnp.float32) + bb_ref[...]
    cm = jnp.dot(u, wc_ref[...], preferred_element_type=jnp.float32) + bc_ref[...]

    a_neg = -jnp.exp(alogT_ref[...])
    da_ref[...] = jnp.exp(dt[:, None, :] * a_neg[None, :, :])
    db_ref[...] = (dt * u)[:, None, :] * bm[:, :, None]

    def step(i, h):
        h = da_ref[i] * h + db_ref[i]
        hs_ref[i] = h
        return h

    h = jax.lax.fori_loop(0, chunk, step, h_ref[...], unroll=16)
    h_ref[...] = h
    y = jnp.sum(hs_ref[...] * cm[:, :, None], axis=1)
    y_ref[0] = y + u * dskip_ref[...]


def _qkv_body(y_ref, w_ref, b_ref, o_ref, *, hd):
    res = jnp.dot(y_ref[...], w_ref[...],
                  preferred_element_type=jnp.float32) + b_ref[...]
    for j in range(res.shape[-1] // hd):
        o_ref[0, j] = res[:, j * hd:(j + 1) * hd]


def _attn_epi_body(q_ref, k_ref, v_ref, y_ref, wo_ref, bo_ref, ang_ref,
                   anb_ref, wout_ref, bout_ref, o_ref, ps_ref, acc_ref, *,
                   scale, hd):
    i = pl.program_id(1)
    acc_ref[...] = jnp.zeros_like(acc_ref)

    def head_pair(hp, _):
        for dh in range(2):
            h = hp * 2 + dh
            q = q_ref[0, h]
            k = k_ref[0, h]
            vv = v_ref[0, h]
            s = jax.lax.dot_general(q, k, (((1,), (1,)), ((), ())),
                                    preferred_element_type=jnp.float32) * scale
            mx = jnp.max(s, axis=-1, keepdims=True)
            p = jnp.exp(s - mx)
            l = jnp.sum(p, axis=-1, keepdims=True)
            o = jnp.dot(p, vv, preferred_element_type=jnp.float32) / l
            wo_h = wo_ref[pl.ds(pl.multiple_of(h * hd, hd), hd), :]
            acc_ref[...] += jnp.dot(o, wo_h, preferred_element_type=jnp.float32)
        return 0

    jax.lax.fori_loop(0, NHEADS // 2, head_pair, 0)
    a = acc_ref[...] + bo_ref[...]
    m = jnp.mean(a, axis=-1, keepdims=True)
    ac = a - m
    v = jnp.mean(ac * ac, axis=-1, keepdims=True)
    ln = ac * jax.lax.rsqrt(v + 1e-5) * ang_ref[...] + anb_ref[...]
    z = y_ref[0] + ln
    out = jnp.dot(z, wout_ref[...], preferred_element_type=jnp.float32) + bout_ref[...]
    o_ref[0] = out
    ps = jnp.sum(out, axis=0, keepdims=True)

    @pl.when(i == 0)
    def _():
        ps_ref[0] = ps

    @pl.when(i != 0)
    def _():
        ps_ref[0] = ps_ref[0] + ps


def _cls_body(pa_ref, pv_ref, g_ref, b_ref, o_ref, *, inv_count):
    s = (pa_ref[0] + pv_ref[0]) * inv_count
    m = jnp.mean(s, axis=-1, keepdims=True)
    sc = s - m
    v = jnp.mean(sc * sc, axis=-1, keepdims=True)
    o_ref[0] = sc * jax.lax.rsqrt(v + 1e-5) * g_ref[...] + b_ref[...]


def kernel(text, audio, video, in_norm_g, in_norm_b, Wi, bi, Wdt, bdt, WB, bB,
           WC, bC, A_log, Dskip, Wqkv, bqkv, Wo, bo, an_g, an_b, Wout, bout,
           on_g, on_b):
    f32 = jnp.float32
    nb, lc, d = text.shape
    n = A_log.shape[1]
    hd = d // NHEADS
    g = 2 * nb
    l = 2 * lc
    rows = g * l

    def fit(feat):
        lm = feat.shape[1]
        if lm < lc:
            feat = jnp.pad(feat, ((0, 0), (0, lc - lm), (0, 0)))
        elif lm > lc:
            feat = feat[:, :lc]
        return feat

    xa = jnp.concatenate([fit(audio), text], axis=1)
    xv = jnp.concatenate([fit(video), text], axis=1)
    x4 = jnp.concatenate([xa, xv], axis=0)

    row = lambda a: a.reshape(1, -1)
    full = lambda shape: pl.BlockSpec(shape, lambda *a: (0,) * len(shape))

    q = 256
    nt = l // q
    y4 = pl.pallas_call(
        functools.partial(_prep_scan_body, chunk=q),
        grid=(g, nt),
        in_specs=[
            pl.BlockSpec((1, q, d), lambda b, t: (b, t, 0)),
            full((1, d)), full((1, d)),
            full((d, d)), full((1, d)),
            full((d, d)), full((1, d)),
            full((d, n)), full((1, n)),
            full((d, n)), full((1, n)),
            full((n, d)), full((1, d)),
        ],
        out_specs=pl.BlockSpec((1, q, d), lambda b, t: (b, t, 0)),
        out_shape=jax.ShapeDtypeStruct((g, l, d), f32),
        scratch_shapes=[
            pltpu.VMEM((n, d), f32),
            pltpu.VMEM((q, n, d), f32),
            pltpu.VMEM((q, n, d), f32),
            pltpu.VMEM((q, n, d), f32),
        ],
        compiler_params=pltpu.CompilerParams(
            dimension_semantics=(pltpu.PARALLEL, pltpu.ARBITRARY),
            vmem_limit_bytes=100 * 1024 * 1024),
    )(x4, row(in_norm_g), row(in_norm_b), Wi.T, row(bi), Wdt.T, row(bdt),
      WB.T, row(bB), WC.T, row(bC), A_log.T, row(Dskip))

    rb = 1024
    nrb = l // rb
    y2 = y4.reshape(rows, d)
    qkvh = pl.pallas_call(
        functools.partial(_qkv_body, hd=hd),
        grid=(rows // rb,),
        in_specs=[
            pl.BlockSpec((rb, d), lambda r: (r, 0)),
            full((d, 3 * d)), full((1, 3 * d)),
        ],
        out_specs=pl.BlockSpec((1, 3 * NHEADS, rb, hd),
                               lambda r: (r // nrb, 0, r % nrb, 0)),
        out_shape=jax.ShapeDtypeStruct((g, 3 * NHEADS, l, hd), f32),
        compiler_params=pltpu.CompilerParams(
            dimension_semantics=(pltpu.PARALLEL,)),
    )(y2, Wqkv.T, row(bqkv))

    qb = 1024
    nqb = l // qb
    out4, psum = pl.pallas_call(
        functools.partial(_attn_epi_body, scale=1.0 / float(hd) ** 0.5, hd=hd),
        grid=(g, nqb),
        in_specs=[
            pl.BlockSpec((1, NHEADS, qb, hd), lambda b, i: (b, 0, i, 0)),
            pl.BlockSpec((1, NHEADS, l, hd), lambda b, i: (b, 1, 0, 0)),
            pl.BlockSpec((1, NHEADS, l, hd), lambda b, i: (b, 2, 0, 0)),
            pl.BlockSpec((1, qb, d), lambda b, i: (b, i, 0)),
            full((d, d)), full((1, d)), full((1, d)), full((1, d)),
            full((d, d)), full((1, d)),
        ],
        out_specs=[
            pl.BlockSpec((1, qb, d), lambda b, i: (b, i, 0)),
            pl.BlockSpec((1, 1, d), lambda b, i: (b, 0, 0)),
        ],
        out_shape=[
            jax.ShapeDtypeStruct((g, l, d), f32),
            jax.ShapeDtypeStruct((g, 1, d), f32),
        ],
        scratch_shapes=[pltpu.VMEM((qb, d), f32)],
        compiler_params=pltpu.CompilerParams(
            dimension_semantics=(pltpu.PARALLEL, pltpu.ARBITRARY),
            vmem_limit_bytes=100 * 1024 * 1024),
    )(qkvh, qkvh, qkvh, y4, Wo.T, row(bo), row(an_g), row(an_b),
      Wout.T, row(bout))

    cls = pl.pallas_call(
        functools.partial(_cls_body, inv_count=0.5 / float(l)),
        grid=(nb,),
        in_specs=[
            pl.BlockSpec((1, 1, d), lambda b: (b, 0, 0)),
            pl.BlockSpec((1, 1, d), lambda b: (b + nb, 0, 0)),
            full((1, d)), full((1, d)),
        ],
        out_specs=pl.BlockSpec((1, 1, d), lambda b: (b, 0, 0)),
        out_shape=jax.ShapeDtypeStruct((nb, 1, d), f32),
        compiler_params=pltpu.CompilerParams(
            dimension_semantics=(pltpu.ARBITRARY,)),
    )(psum, psum, row(on_g), row(on_b))

    return cls.reshape(nb, d), out4[:nb], out4[nb:]
